# baseline (speedup 1.0000x reference)
import functools

import jax
import jax.numpy as jnp
from jax import lax
from jax.experimental import pallas as pl
from jax.experimental.pallas import tpu as pltpu
from jax.experimental.pallas import tpu_sc as plsc

_BIG = 1 << 30
_BB = 16
_B_SC = 64
_NW = 32
_CH = 32768
_U = 8


def _argmax_body(x_ref, idx_ref):
    bb, h, w = x_ref.shape
    ciota = lax.broadcasted_iota(jnp.int32, (1, w), 1)
    for s in range(bb):
        rowmax = jnp.max(x_ref[s], axis=1, keepdims=True)
        m = jnp.max(rowmax)
        riota = lax.broadcasted_iota(jnp.int32, (h, 1), 0)
        row = jnp.min(jnp.where(rowmax == m, riota, _BIG))
        rv = x_ref[s, pl.ds(row, 1), :]
        col = jnp.min(jnp.where(rv == m, ciota, _BIG))
        idx_ref[s, 0, 0] = row * w + col


def _tc_argmax(x3, b_tc):
    b, h, w = x3.shape
    return pl.pallas_call(
        _argmax_body,
        grid=(b_tc // _BB,),
        in_specs=[pl.BlockSpec((_BB, h, w), lambda i: (i, 0, 0))],
        out_specs=pl.BlockSpec((_BB, 1, 1), lambda i: (i, 0, 0),
                               memory_space=pltpu.SMEM),
        out_shape=jax.ShapeDtypeStruct((b_tc, 1, 1), jnp.int32),
        compiler_params=pltpu.CompilerParams(
            dimension_semantics=("arbitrary",)),
    )(x3)


def _permute16(x, perm):
    dnums = lax.GatherDimensionNumbers(
        offset_dims=(), collapsed_slice_dims=(0,), start_index_map=(0,))
    return lax.gather(x, perm[:, None], dnums, slice_sizes=(1,),
                      mode=lax.GatherScatterMode.PROMISE_IN_BOUNDS)


def _lexmerge(va, pa, vb, pb):
    better = (vb > va) | ((vb == va) & (pb < pa))
    return jnp.where(better, vb, va), jnp.where(better, pb, pa)


def _make_sc_argmax(b, h, w, b_tc):
    per_tile = (b - b_tc) // _NW
    rb = _CH // w
    nchunks = h // rb
    gpr = w // 16
    it_in = gpr // _U
    mesh = plsc.VectorSubcoreMesh(core_axis_name="c", subcore_axis_name="s")

    @functools.partial(
        pl.kernel,
        mesh=mesh,
        out_type=jax.ShapeDtypeStruct((_NW * 16,), jnp.int32),
        scratch_types=[pltpu.VMEM((rb, w), jnp.float32),
                       pltpu.VMEM((rb, w), jnp.float32),
                       pltpu.VMEM((16,), jnp.int32),
                       pltpu.SemaphoreType.DMA,
                       pltpu.SemaphoreType.DMA],
    )
    def sc_argmax(x_hbm, out_hbm, buf0, buf1, res_v, sem0, sem1):
        wid = lax.axis_index("s") * 2 + lax.axis_index("c")
        bufs = (buf0, buf1)
        sems = (sem0, sem1)
        liota = lax.broadcasted_iota(jnp.int32, (16,), 0)
        res = jnp.zeros((16,), jnp.int32)

        for j in range(per_tile):
            row = b_tc + wid * per_tile + j
            ms = tuple(jnp.full((16,), -jnp.inf, jnp.float32)
                       for _ in range(_U))
            ps = tuple(jnp.zeros((16,), jnp.int32) for _ in range(_U))
            carry = (ms, ps)
            handles = [None, None]
            handles[0] = pltpu.async_copy(
                x_hbm.at[row, pl.ds(0, rb), :], buf0, sem0)
            for c in range(nchunks):
                if c + 1 < nchunks:
                    nb = (c + 1) % 2
                    handles[nb] = pltpu.async_copy(
                        x_hbm.at[row, pl.ds((c + 1) * rb, rb), :],
                        bufs[nb], sems[nb])
                handles[c % 2].wait()
                cur = bufs[c % 2]
                base = c * rb * it_in

                def outer(rr, carry, cur=cur, base=base):
                    cms, cps = list(carry[0]), list(carry[1])
                    rbase = base + rr * it_in
                    for ci in range(it_in):
                        pv = jnp.broadcast_to(rbase + ci, (16,))
                        for u in range(_U):
                            v = cur[rr, pl.ds(ci * (16 * _U) + u * 16, 16)]
                            g = v > cms[u]
                            cms[u] = jnp.where(g, v, cms[u])
                            cps[u] = jnp.where(g, pv, cps[u])
                    return tuple(cms), tuple(cps)

                carry = lax.fori_loop(0, rb, outer, carry)

            ms, ps = carry
            bv = ms[0]
            bp = ps[0] * (16 * _U) + liota
            for u in range(1, _U):
                bv, bp = _lexmerge(bv, bp, ms[u],
                                   ps[u] * (16 * _U) + (u * 16) + liota)
            for d in (8, 4, 2, 1):
                perm = liota ^ d
                v2 = _permute16(bv, perm)
                p2 = _permute16(bp, perm)
                bv, bp = _lexmerge(bv, bp, v2, p2)
            res = jnp.where(liota == j, bp, res)

        res_v[...] = res
        pltpu.sync_copy(res_v, out_hbm.at[pl.ds(wid * 16, 16)])

    return sc_argmax


def _make_sc_gather(b):
    chunk = 8
    nworkers = b // chunk
    mesh = plsc.VectorSubcoreMesh(core_axis_name="c", subcore_axis_name="s")

    @functools.partial(
        pl.kernel,
        mesh=mesh,
        out_type=[jax.ShapeDtypeStruct((b,), jnp.float32),
                  jax.ShapeDtypeStruct((b,), jnp.float32)],
        scratch_types=[pltpu.VMEM((chunk,), jnp.int32),
                       pltpu.VMEM((chunk,), jnp.float32),
                       pltpu.VMEM((chunk,), jnp.float32),
                       pltpu.SemaphoreType.DMA],
    )
    def gather_k(idx_hbm, gx_hbm, gy_hbm, ox_hbm, oy_hbm,
                 idx_v, x_v, y_v, sem):
        wid = lax.axis_index("s") * 2 + lax.axis_index("c")

        @pl.when(wid < nworkers)
        def _():
            base = wid * chunk
            pltpu.sync_copy(idx_hbm.at[pl.ds(base, chunk)], idx_v)
            pltpu.async_copy(gx_hbm.at[idx_v], x_v, sem).wait()
            pltpu.async_copy(gy_hbm.at[idx_v], y_v, sem).wait()
            pltpu.sync_copy(x_v, ox_hbm.at[pl.ds(base, chunk)])
            pltpu.sync_copy(y_v, oy_hbm.at[pl.ds(base, chunk)])

    return gather_k


def kernel(input, grid_x, grid_y):
    b = input.shape[0]
    h, w = input.shape[2], input.shape[3]
    n = h * w
    b_tc = b - _B_SC
    per_tile = _B_SC // _NW
    x3 = input.reshape(b, h, w)
    tc_idx = _tc_argmax(x3, b_tc).reshape(b_tc)
    sc_out = _make_sc_argmax(b, h, w, b_tc)(x3)
    sc_idx = sc_out.reshape(_NW, 16)[:, :per_tile].reshape(_B_SC)
    idx = jnp.concatenate((tc_idx, sc_idx))
    gx = grid_x.reshape(n)
    gy = grid_y.reshape(n)
    ox, oy = _make_sc_gather(b)(idx, gx, gy)
    return jnp.concatenate((ox[:, None], oy[:, None]), axis=1)


# speedup vs baseline: 1.1220x; 1.1220x over previous
import functools

import jax
import jax.numpy as jnp
from jax import lax
from jax.experimental import pallas as pl
from jax.experimental.pallas import tpu as pltpu
from jax.experimental.pallas import tpu_sc as plsc

_BIG = 1 << 30
_BB = 16
_B_SC = 48
_NW = 32
_CH = 16384
_NBUF = 4
_U = 8


def _argmax_body(x_ref, idx_ref):
    bb, h, w = x_ref.shape
    ciota = lax.broadcasted_iota(jnp.int32, (1, w), 1)
    for s in range(bb):
        rowmax = jnp.max(x_ref[s], axis=1, keepdims=True)
        m = jnp.max(rowmax)
        riota = lax.broadcasted_iota(jnp.int32, (h, 1), 0)
        row = jnp.min(jnp.where(rowmax == m, riota, _BIG))
        rv = x_ref[s, pl.ds(row, 1), :]
        col = jnp.min(jnp.where(rv == m, ciota, _BIG))
        idx_ref[s, 0, 0] = row * w + col


def _tc_argmax(x3, b_tc):
    b, h, w = x3.shape
    return pl.pallas_call(
        _argmax_body,
        grid=(b_tc // _BB,),
        in_specs=[pl.BlockSpec((_BB, h, w), lambda i: (i, 0, 0))],
        out_specs=pl.BlockSpec((_BB, 1, 1), lambda i: (i, 0, 0),
                               memory_space=pltpu.SMEM),
        out_shape=jax.ShapeDtypeStruct((b_tc, 1, 1), jnp.int32),
        compiler_params=pltpu.CompilerParams(
            dimension_semantics=("arbitrary",)),
    )(x3)


def _permute16(x, perm):
    dnums = lax.GatherDimensionNumbers(
        offset_dims=(), collapsed_slice_dims=(0,), start_index_map=(0,))
    return lax.gather(x, perm[:, None], dnums, slice_sizes=(1,),
                      mode=lax.GatherScatterMode.PROMISE_IN_BOUNDS)


def _lexmerge(va, pa, vb, pb):
    better = (vb > va) | ((vb == va) & (pb < pa))
    return jnp.where(better, vb, va), jnp.where(better, pb, pa)


def _make_sc_argmax(b, h, w, b_tc):
    b_sc = b - b_tc
    rounds = -(-b_sc // _NW)
    rb = _CH // w
    nchunks = h // rb
    gpr = w // 16
    it_in = gpr // _U
    mesh = plsc.VectorSubcoreMesh(core_axis_name="c", subcore_axis_name="s")

    @functools.partial(
        pl.kernel,
        mesh=mesh,
        out_type=jax.ShapeDtypeStruct((_NW * 16,), jnp.int32),
        scratch_types=([pltpu.VMEM((rb, w), jnp.float32)] * _NBUF
                       + [pltpu.VMEM((16,), jnp.int32)]
                       + [pltpu.SemaphoreType.DMA] * _NBUF),
    )
    def sc_argmax(x_hbm, out_hbm, *refs):
        bufs = refs[:_NBUF]
        res_v = refs[_NBUF]
        sems = refs[_NBUF + 1:]
        wid = lax.axis_index("s") * 2 + lax.axis_index("c")
        liota = lax.broadcasted_iota(jnp.int32, (16,), 0)
        res_v[...] = jnp.zeros((16,), jnp.int32)

        def one_sample(j):
            row = b_tc + 32 * j + wid
            ms = tuple(jnp.full((16,), -jnp.inf, jnp.float32)
                       for _ in range(_U))
            ps = tuple(jnp.zeros((16,), jnp.int32) for _ in range(_U))
            carry = (ms, ps)
            handles = [None] * _NBUF
            for p in range(_NBUF - 1):
                handles[p] = pltpu.async_copy(
                    x_hbm.at[row, pl.ds(p * rb, rb), :], bufs[p], sems[p])
            for c in range(nchunks):
                if c + _NBUF - 1 < nchunks:
                    nb = (c + _NBUF - 1) % _NBUF
                    handles[nb] = pltpu.async_copy(
                        x_hbm.at[row, pl.ds((c + _NBUF - 1) * rb, rb), :],
                        bufs[nb], sems[nb])
                handles[c % _NBUF].wait()
                cur = bufs[c % _NBUF]
                base = c * rb * it_in

                def outer(rr, carry, cur=cur, base=base):
                    cms, cps = list(carry[0]), list(carry[1])
                    rbase = base + rr * it_in
                    for ci in range(it_in):
                        pv = jnp.broadcast_to(rbase + ci, (16,))
                        for u in range(_U):
                            v = cur[rr, pl.ds(ci * (16 * _U) + u * 16, 16)]
                            g = v > cms[u]
                            cms[u] = jnp.maximum(v, cms[u])
                            cps[u] = jnp.where(g, pv, cps[u])
                    return tuple(cms), tuple(cps)

                carry = lax.fori_loop(0, rb, outer, carry)

            ms, ps = carry
            bv = ms[0]
            bp = ps[0] * (16 * _U) + liota
            for u in range(1, _U):
                bv, bp = _lexmerge(bv, bp, ms[u],
                                   ps[u] * (16 * _U) + (u * 16) + liota)
            for d in (8, 4, 2, 1):
                perm = liota ^ d
                v2 = _permute16(bv, perm)
                p2 = _permute16(bp, perm)
                bv, bp = _lexmerge(bv, bp, v2, p2)
            res_v[...] = jnp.where(liota == j, bp, res_v[...])

        for j in range(rounds):
            nact = min(_NW, b_sc - _NW * j)
            if nact == _NW:
                one_sample(j)
            else:
                pl.when(wid < nact)(lambda j=j: one_sample(j))

        pltpu.sync_copy(res_v, out_hbm.at[pl.ds(wid * 16, 16)])

    return sc_argmax


def _make_sc_gather(b, b_tc):
    n_tc_w = b_tc // 16
    n_w = b // 16
    mesh = plsc.VectorSubcoreMesh(core_axis_name="c", subcore_axis_name="s")

    @functools.partial(
        pl.kernel,
        mesh=mesh,
        out_type=jax.ShapeDtypeStruct((2 * b,), jnp.float32),
        scratch_types=[pltpu.VMEM((16,), jnp.int32),
                       pltpu.VMEM((16,), jnp.float32),
                       pltpu.VMEM((16,), jnp.float32),
                       pltpu.VMEM((16,), jnp.int32),
                       pltpu.SemaphoreType.DMA],
    )
    def gather_k(tci_hbm, sci_hbm, gx_hbm, gy_hbm, out_hbm,
                 idx_v, x_v, y_v, pos_v, sem):
        wid = lax.axis_index("s") * 2 + lax.axis_index("c")
        liota = lax.broadcasted_iota(jnp.int32, (16,), 0)

        def finish(w):
            pltpu.async_copy(gx_hbm.at[idx_v], x_v, sem).wait()
            pltpu.async_copy(gy_hbm.at[idx_v], y_v, sem).wait()
            pltpu.sync_copy(x_v, out_hbm.at[pl.ds(w * 16, 16)])
            pltpu.sync_copy(y_v, out_hbm.at[pl.ds(b + w * 16, 16)])

        @pl.when(wid < n_tc_w)
        def _():
            pltpu.sync_copy(tci_hbm.at[pl.ds(wid * 16, 16)], idx_v)
            finish(wid)

        @pl.when((wid >= n_tc_w) & (wid < n_w))
        def _():
            k = (wid - n_tc_w) * 16 + liota
            pos_v[...] = ((k & 31) * 16) + (k >> 5)
            pltpu.async_copy(sci_hbm.at[pos_v], idx_v, sem).wait()
            finish(wid)

    return gather_k


def kernel(input, grid_x, grid_y):
    b = input.shape[0]
    h, w = input.shape[2], input.shape[3]
    n = h * w
    b_tc = b - _B_SC
    x3 = input.reshape(b, h, w)
    tc_idx = _tc_argmax(x3, b_tc).reshape(b_tc)
    sc_out = _make_sc_argmax(b, h, w, b_tc)(x3)
    gx = grid_x.reshape(n)
    gy = grid_y.reshape(n)
    xy = _make_sc_gather(b, b_tc)(tc_idx, sc_out, gx, gy)
    return xy.reshape(2, b).T
